# baseline (speedup 1.0000x reference)
import functools

import jax
import jax.numpy as jnp
from jax.experimental import pallas as pl
from jax.experimental.pallas import tpu as pltpu


def _router_body(x_ref, w0_ref, b0_ref, w1_ref, b1_ref, w2_ref, b2_ref,
                 out_ref, *, s_t: int):
    s = pl.program_id(1)
    xt = x_ref[0].astype(jnp.bfloat16)
    h = jnp.dot(xt, w0_ref[...], preferred_element_type=jnp.float32)
    h = jnp.maximum(h + b0_ref[...], 0.0).astype(jnp.bfloat16)
    h = jnp.dot(h, w1_ref[...], preferred_element_type=jnp.float32)
    h = jnp.maximum(h + b1_ref[...], 0.0)
    logits = jnp.dot(h, w2_ref[...], preferred_element_type=jnp.float32)
    out_ref[0, pl.ds(s * s_t, s_t), :] = logits + b2_ref[...]

    @pl.when(s == pl.num_programs(1) - 1)
    def _softmax():
        lg = out_ref[0]
        m = jnp.max(lg, axis=0, keepdims=True)
        e = jnp.exp(lg - m)
        out_ref[0] = e / jnp.sum(e, axis=0, keepdims=True)


@jax.jit
def kernel(x, W0, b0, W1, b1, W2, b2):
    B, S, D = x.shape
    H0 = W0.shape[1]
    H1 = W1.shape[1]
    E = W2.shape[1]
    S_T = 512
    grid = (B, S // S_T)

    b0r = b0.reshape(1, H0)
    b1r = b1.reshape(1, H1)
    b2r = b2.reshape(1, E)
    W0h = W0.astype(jnp.bfloat16)
    W1h = W1.astype(jnp.bfloat16)

    return pl.pallas_call(
        functools.partial(_router_body, s_t=S_T),
        grid=grid,
        in_specs=[
            pl.BlockSpec((1, S_T, D), lambda b, s: (b, s, 0)),
            pl.BlockSpec((D, H0), lambda b, s: (0, 0)),
            pl.BlockSpec((1, H0), lambda b, s: (0, 0)),
            pl.BlockSpec((H0, H1), lambda b, s: (0, 0)),
            pl.BlockSpec((1, H1), lambda b, s: (0, 0)),
            pl.BlockSpec((H1, E), lambda b, s: (0, 0)),
            pl.BlockSpec((1, E), lambda b, s: (0, 0)),
        ],
        out_specs=pl.BlockSpec((1, S, E), lambda b, s: (b, 0, 0)),
        out_shape=jax.ShapeDtypeStruct((B, S, E), jnp.float32),
        compiler_params=pltpu.CompilerParams(
            dimension_semantics=("arbitrary", "arbitrary")
        ),
    )(x, W0h, b0r, W1h, b1r, W2, b2r)


# speedup vs baseline: 1.0509x; 1.0509x over previous
import functools

import jax
import jax.numpy as jnp
from jax.experimental import pallas as pl
from jax.experimental.pallas import tpu as pltpu


def _router_body(x_ref, w0_ref, b0_ref, w1_ref, b1_ref, w2_ref, b2_ref,
                 out_ref, *, s_t: int):
    s = pl.program_id(1)
    xt = x_ref[0]
    h = jnp.dot(xt, w0_ref[...], preferred_element_type=jnp.float32)
    h = jnp.maximum(h + b0_ref[...], 0.0)
    h = jnp.dot(h, w1_ref[...], preferred_element_type=jnp.float32)
    h = jnp.maximum(h + b1_ref[...], 0.0)
    logits = jnp.dot(h, w2_ref[...], preferred_element_type=jnp.float32)
    out_ref[0, pl.ds(s * s_t, s_t), :] = logits + b2_ref[...]

    @pl.when(s == pl.num_programs(1) - 1)
    def _softmax():
        lg = out_ref[0]
        m = jnp.max(lg, axis=0, keepdims=True)
        e = jnp.exp(lg - m)
        out_ref[0] = e / jnp.sum(e, axis=0, keepdims=True)


@jax.jit
def kernel(x, W0, b0, W1, b1, W2, b2):
    B, S, D = x.shape
    H0 = W0.shape[1]
    H1 = W1.shape[1]
    E = W2.shape[1]
    S_T = 512
    grid = (B, S // S_T)

    b0r = b0.reshape(1, H0)
    b1r = b1.reshape(1, H1)
    b2r = b2.reshape(1, E)

    return pl.pallas_call(
        functools.partial(_router_body, s_t=S_T),
        grid=grid,
        in_specs=[
            pl.BlockSpec((1, S_T, D), lambda b, s: (b, s, 0)),
            pl.BlockSpec((D, H0), lambda b, s: (0, 0)),
            pl.BlockSpec((1, H0), lambda b, s: (0, 0)),
            pl.BlockSpec((H0, H1), lambda b, s: (0, 0)),
            pl.BlockSpec((1, H1), lambda b, s: (0, 0)),
            pl.BlockSpec((H1, E), lambda b, s: (0, 0)),
            pl.BlockSpec((1, E), lambda b, s: (0, 0)),
        ],
        out_specs=pl.BlockSpec((1, S, E), lambda b, s: (b, 0, 0)),
        out_shape=jax.ShapeDtypeStruct((B, S, E), jnp.float32),
        compiler_params=pltpu.CompilerParams(
            dimension_semantics=("arbitrary", "arbitrary")
        ),
    )(x, W0, b0r, W1, b1r, W2, b2r)


# speedup vs baseline: 1.1061x; 1.0525x over previous
import functools

import jax
import jax.numpy as jnp
from jax.experimental import pallas as pl
from jax.experimental.pallas import tpu as pltpu


def _router_body(x_ref, w0_ref, b0_ref, w1_ref, b1_ref, w2_ref, b2_ref,
                 out_ref, *, s_t: int):
    s = pl.program_id(1)
    xt = x_ref[0]
    h = jnp.dot(xt, w0_ref[...], preferred_element_type=jnp.float32)
    h = jnp.maximum(h + b0_ref[...], 0.0)
    h = jnp.dot(h, w1_ref[...], preferred_element_type=jnp.float32)
    h = jnp.maximum(h + b1_ref[...], 0.0)
    logits = jnp.dot(h, w2_ref[...], preferred_element_type=jnp.float32)
    out_ref[0, pl.ds(s * s_t, s_t), :] = logits + b2_ref[...]

    @pl.when(s == pl.num_programs(1) - 1)
    def _softmax():
        lg = out_ref[0]
        m = jnp.max(lg, axis=0, keepdims=True)
        e = jnp.exp(lg - m)
        out_ref[0] = e / jnp.sum(e, axis=0, keepdims=True)


@jax.jit
def kernel(x, W0, b0, W1, b1, W2, b2):
    B, S, D = x.shape
    H0 = W0.shape[1]
    H1 = W1.shape[1]
    E = W2.shape[1]
    S_T = 1024
    grid = (B, S // S_T)

    b0r = b0.reshape(1, H0)
    b1r = b1.reshape(1, H1)
    b2r = b2.reshape(1, E)

    return pl.pallas_call(
        functools.partial(_router_body, s_t=S_T),
        grid=grid,
        in_specs=[
            pl.BlockSpec((1, S_T, D), lambda b, s: (b, s, 0)),
            pl.BlockSpec((D, H0), lambda b, s: (0, 0)),
            pl.BlockSpec((1, H0), lambda b, s: (0, 0)),
            pl.BlockSpec((H0, H1), lambda b, s: (0, 0)),
            pl.BlockSpec((1, H1), lambda b, s: (0, 0)),
            pl.BlockSpec((H1, E), lambda b, s: (0, 0)),
            pl.BlockSpec((1, E), lambda b, s: (0, 0)),
        ],
        out_specs=pl.BlockSpec((1, S, E), lambda b, s: (b, 0, 0)),
        out_shape=jax.ShapeDtypeStruct((B, S, E), jnp.float32),
        compiler_params=pltpu.CompilerParams(
            dimension_semantics=("arbitrary", "arbitrary")
        ),
    )(x, W0, b0r, W1, b1r, W2, b2r)


# speedup vs baseline: 1.1066x; 1.0005x over previous
import functools

import jax
import jax.numpy as jnp
from jax.experimental import pallas as pl
from jax.experimental.pallas import tpu as pltpu


def _router_body(x_ref, w0_ref, b0_ref, w1_ref, b1_ref, w2_ref, b2_ref,
                 out_ref, *, s_t: int):
    s = pl.program_id(1)
    xt = x_ref[0]
    h = jnp.dot(xt, w0_ref[...], preferred_element_type=jnp.float32)
    h = jnp.maximum(h + b0_ref[...], 0.0)
    h = jnp.dot(h, w1_ref[...], preferred_element_type=jnp.float32)
    h = jnp.maximum(h + b1_ref[...], 0.0)
    logits = jnp.dot(h, w2_ref[...], preferred_element_type=jnp.float32)
    out_ref[0, pl.ds(s * s_t, s_t), :] = logits + b2_ref[...]

    @pl.when(s == pl.num_programs(1) - 1)
    def _softmax():
        lg = out_ref[0]
        m = jnp.max(lg, axis=0, keepdims=True)
        e = jnp.exp(lg - m)
        out_ref[0] = e / jnp.sum(e, axis=0, keepdims=True)


@jax.jit
def kernel(x, W0, b0, W1, b1, W2, b2):
    B, S, D = x.shape
    H0 = W0.shape[1]
    H1 = W1.shape[1]
    E = W2.shape[1]
    S_T = 2048
    grid = (B, S // S_T)

    b0r = b0.reshape(1, H0)
    b1r = b1.reshape(1, H1)
    b2r = b2.reshape(1, E)

    return pl.pallas_call(
        functools.partial(_router_body, s_t=S_T),
        grid=grid,
        in_specs=[
            pl.BlockSpec((1, S_T, D), lambda b, s: (b, s, 0)),
            pl.BlockSpec((D, H0), lambda b, s: (0, 0)),
            pl.BlockSpec((1, H0), lambda b, s: (0, 0)),
            pl.BlockSpec((H0, H1), lambda b, s: (0, 0)),
            pl.BlockSpec((1, H1), lambda b, s: (0, 0)),
            pl.BlockSpec((H1, E), lambda b, s: (0, 0)),
            pl.BlockSpec((1, E), lambda b, s: (0, 0)),
        ],
        out_specs=pl.BlockSpec((1, S, E), lambda b, s: (b, 0, 0)),
        out_shape=jax.ShapeDtypeStruct((B, S, E), jnp.float32),
        compiler_params=pltpu.CompilerParams(
            dimension_semantics=("arbitrary", "arbitrary")
        ),
    )(x, W0, b0r, W1, b1r, W2, b2r)


# speedup vs baseline: 1.1068x; 1.0001x over previous
import functools

import jax
import jax.numpy as jnp
from jax.experimental import pallas as pl
from jax.experimental.pallas import tpu as pltpu


def _router_body(x_ref, w0_ref, b0_ref, w1_ref, b1_ref, w2_ref, b2_ref,
                 out_ref, *, s_t: int):
    s = pl.program_id(1)
    xt = x_ref[0]
    h = jnp.dot(xt, w0_ref[...], preferred_element_type=jnp.float32)
    h = jnp.maximum(h + b0_ref[...], 0.0)
    h = jnp.dot(h, w1_ref[...], preferred_element_type=jnp.float32)
    h = jnp.maximum(h + b1_ref[...], 0.0)
    logits = jnp.dot(h, w2_ref[...], preferred_element_type=jnp.float32)
    out_ref[0, pl.ds(s * s_t, s_t), :] = logits + b2_ref[...]

    @pl.when(s == pl.num_programs(1) - 1)
    def _softmax():
        lg = out_ref[0]
        m = jnp.max(lg, axis=0, keepdims=True)
        e = jnp.exp(lg - m)
        out_ref[0] = e / jnp.sum(e, axis=0, keepdims=True)


@jax.jit
def kernel(x, W0, b0, W1, b1, W2, b2):
    B, S, D = x.shape
    H0 = W0.shape[1]
    H1 = W1.shape[1]
    E = W2.shape[1]
    S_T = 1024
    grid = (B, S // S_T)

    b0r = b0.reshape(1, H0)
    b1r = b1.reshape(1, H1)
    b2r = b2.reshape(1, E)

    return pl.pallas_call(
        functools.partial(_router_body, s_t=S_T),
        grid=grid,
        in_specs=[
            pl.BlockSpec((1, S_T, D), lambda b, s: (b, s, 0)),
            pl.BlockSpec((D, H0), lambda b, s: (0, 0)),
            pl.BlockSpec((1, H0), lambda b, s: (0, 0)),
            pl.BlockSpec((H0, H1), lambda b, s: (0, 0)),
            pl.BlockSpec((1, H1), lambda b, s: (0, 0)),
            pl.BlockSpec((H1, E), lambda b, s: (0, 0)),
            pl.BlockSpec((1, E), lambda b, s: (0, 0)),
        ],
        out_specs=pl.BlockSpec((1, S, E), lambda b, s: (b, 0, 0)),
        out_shape=jax.ShapeDtypeStruct((B, S, E), jnp.float32),
        compiler_params=pltpu.CompilerParams(
            dimension_semantics=("parallel", "arbitrary")
        ),
    )(x, W0, b0r, W1, b1r, W2, b2r)
